# Initial kernel scaffold; baseline (speedup 1.0000x reference)
import jax
import jax.numpy as jnp
from jax.experimental import pallas as pl


def kernel(weight):
    raise NotImplementedError("write your pallas kernel here")


# speedup vs baseline: 1.7398x; 1.7398x over previous
import jax
import jax.numpy as jnp
from jax.experimental import pallas as pl
from jax.experimental.pallas import tpu as pltpu

N = 8192
D = 512
BR = 512
NI = N // BR


def _emb_loss_kernel(w_blk_ref, w_full_ref, out_ref, m_scratch, sqrow_scratch):
    i = pl.program_id(0)

    w_blk = w_blk_ref[...]
    w_full = w_full_ref[...]

    @pl.when(i == 0)
    def _():
        w2 = w_full * w_full
        sqrow_scratch[...] = jax.lax.dot_general(
            jnp.ones((1, D), jnp.float32), w2,
            dimension_numbers=(((1,), (1,)), ((), ())),
            preferred_element_type=jnp.float32,
        )

    g = jax.lax.dot_general(
        w_blk, w_full,
        dimension_numbers=(((1,), (1,)), ((), ())),
        preferred_element_type=jnp.float32,
    )

    t = sqrow_scratch[...] - 2.0 * g

    col = jax.lax.broadcasted_iota(jnp.int32, (BR, N), 1)
    row = jax.lax.broadcasted_iota(jnp.int32, (BR, N), 0) + i * BR
    t = jnp.where(col == row, jnp.inf, t)

    m = jnp.min(t, axis=1, keepdims=True)
    m_scratch[pl.ds(i * BR, BR), :] = m

    @pl.when(i == NI - 1)
    def _():
        sq_col = jnp.sum(w_full * w_full, axis=1, keepdims=True)
        min_d2 = sq_col + m_scratch[...]
        d = jnp.sqrt(jnp.maximum(min_d2, 1e-12))
        mean = jnp.sum(d) / N
        kept = jnp.where(d > mean, 0.0, d)
        loss = -(jnp.sum(kept) / N)
        out_ref[0] = loss
        out_ref[1] = mean


def kernel(weight):
    out = pl.pallas_call(
        _emb_loss_kernel,
        grid=(NI,),
        in_specs=[
            pl.BlockSpec((BR, D), lambda i: (i, 0)),
            pl.BlockSpec((N, D), lambda i: (0, 0)),
        ],
        out_specs=pl.BlockSpec(memory_space=pltpu.SMEM),
        out_shape=jax.ShapeDtypeStruct((2,), jnp.float32),
        scratch_shapes=[
            pltpu.VMEM((N, 1), jnp.float32),
            pltpu.VMEM((1, N), jnp.float32),
        ],
    )(weight, weight)
    return out


# speedup vs baseline: 2.4114x; 1.3860x over previous
import jax
import jax.numpy as jnp
from jax.experimental import pallas as pl
from jax.experimental.pallas import tpu as pltpu

N = 8192
D = 512
BR = 512
NI = N // BR


def _emb_loss_kernel(w_ref, out_ref, m_scratch):
    w_full = w_ref[...]
    w2 = w_full * w_full

    sq_row = jax.lax.dot_general(
        jnp.ones((1, D), jnp.float32), w2,
        dimension_numbers=(((1,), (1,)), ((), ())),
        preferred_element_type=jnp.float32,
    )

    wbf = w_full.astype(jnp.bfloat16)

    r = jax.lax.broadcasted_iota(jnp.int32, (BR, BR), 0)
    c = jax.lax.broadcasted_iota(jnp.int32, (BR, BR), 1)
    eye_inf = jnp.where(r == c, jnp.inf, 0.0).astype(jnp.float32)

    for i in range(NI):
        lhs = (w_full[i * BR:(i + 1) * BR, :] * -2.0).astype(jnp.bfloat16)
        g = jax.lax.dot_general(
            lhs, wbf,
            dimension_numbers=(((1,), (1,)), ((), ())),
            preferred_element_type=jnp.float32,
        )
        t = g + sq_row
        lo, hi = i * BR, (i + 1) * BR
        pieces = []
        if lo > 0:
            pieces.append(t[:, :lo])
        pieces.append(t[:, lo:hi] + eye_inf)
        if hi < N:
            pieces.append(t[:, hi:])
        t = jnp.concatenate(pieces, axis=1) if len(pieces) > 1 else pieces[0]
        m_scratch[i * BR:(i + 1) * BR, :] = jnp.min(t, axis=1, keepdims=True)

    sq_col = jax.lax.dot_general(
        w2, jnp.ones((D, 1), jnp.float32),
        dimension_numbers=(((1,), (0,)), ((), ())),
        preferred_element_type=jnp.float32,
    )
    min_d2 = sq_col + m_scratch[...]
    d = jnp.sqrt(jnp.maximum(min_d2, 1e-12))
    mean = jnp.sum(d) / N
    kept = jnp.where(d > mean, 0.0, d)
    loss = -(jnp.sum(kept) / N)
    out_ref[0] = loss
    out_ref[1] = mean


def kernel(weight):
    out = pl.pallas_call(
        _emb_loss_kernel,
        in_specs=[pl.BlockSpec((N, D), lambda: (0, 0))],
        out_specs=pl.BlockSpec(memory_space=pltpu.SMEM),
        out_shape=jax.ShapeDtypeStruct((2,), jnp.float32),
        scratch_shapes=[pltpu.VMEM((N, 1), jnp.float32)],
    )(weight)
    return out
